# Initial kernel scaffold; baseline (speedup 1.0000x reference)
import jax
import jax.numpy as jnp
from jax.experimental import pallas as pl


def kernel(x, gate_w, w1, w2):
    raise NotImplementedError("write your pallas kernel here")


# speedup vs baseline: 1.3621x; 1.3621x over previous
import functools

import jax
import jax.numpy as jnp
from jax.experimental import pallas as pl
from jax.experimental.pallas import tpu as pltpu

D_MODEL = 768
D_FF = 3072
NUM_EXPERTS = 8
TOP_K = 2

_NT = 1
_NF = 2


def _moe_body(x_ref, gw_ref, w1_ref, w2_ref, out_ref, comb_ref):
    e = pl.program_id(1)
    f = pl.program_id(2)

    @pl.when((e == 0) & (f == 0))
    def _router():
        xt = x_ref[...]
        logits = jax.lax.dot_general(
            xt.astype(jnp.bfloat16), gw_ref[...].astype(jnp.bfloat16),
            (((1,), (1,)), ((), ())),
            preferred_element_type=jnp.float32)
        m = jnp.max(logits, axis=-1, keepdims=True)
        ex = jnp.exp(logits - m)
        scores = ex / jnp.sum(ex, axis=-1, keepdims=True)
        iota = jax.lax.broadcasted_iota(jnp.int32, scores.shape, 1)
        m1 = jnp.max(scores, axis=-1, keepdims=True)
        i1 = jnp.min(jnp.where(scores == m1, iota, NUM_EXPERTS), axis=-1,
                     keepdims=True)
        oh1 = iota == i1
        s2 = jnp.where(oh1, -1.0, scores)
        m2 = jnp.max(s2, axis=-1, keepdims=True)
        i2 = jnp.min(jnp.where(s2 == m2, iota, NUM_EXPERTS), axis=-1,
                     keepdims=True)
        oh2 = iota == i2
        denom = m1 + m2 + 1e-9
        comb_ref[...] = (jnp.where(oh1, m1 / denom, 0.0)
                         + jnp.where(oh2, m2 / denom, 0.0))
        out_ref[...] = jnp.zeros_like(out_ref)

    xt = x_ref[...]
    h = jax.lax.dot_general(
        xt, w1_ref[0], (((1,), (1,)), ((), ())),
        preferred_element_type=jnp.float32)
    h = jnp.maximum(h, 0.0)
    eo = jax.lax.dot_general(
        h, w2_ref[0], (((1,), (1,)), ((), ())),
        preferred_element_type=jnp.float32)
    iota = jax.lax.broadcasted_iota(jnp.int32, comb_ref.shape, 1)
    ce = jnp.sum(jnp.where(iota == e, comb_ref[...], 0.0), axis=-1,
                 keepdims=True)
    out_ref[...] += ce * eo


@jax.jit
def kernel(x, gate_w, w1, w2):
    B, T, H = x.shape
    N = B * T
    rows = N // _NT
    xf = x.reshape(N, H)
    fchunk = D_FF // _NF
    out = pl.pallas_call(
        _moe_body,
        grid=(_NT, NUM_EXPERTS, _NF),
        in_specs=[
            pl.BlockSpec((rows, H), lambda n, e, f: (n, 0)),
            pl.BlockSpec((NUM_EXPERTS, H), lambda n, e, f: (0, 0)),
            pl.BlockSpec((1, fchunk, H), lambda n, e, f: (e, f, 0)),
            pl.BlockSpec((1, H, fchunk), lambda n, e, f: (e, 0, f)),
        ],
        out_specs=pl.BlockSpec((rows, H), lambda n, e, f: (n, 0)),
        out_shape=jax.ShapeDtypeStruct((N, H), jnp.float32),
        scratch_shapes=[pltpu.VMEM((rows, NUM_EXPERTS), jnp.float32)],
    )(xf, gate_w, w1, w2)
    return out.reshape(B, T, H)


# speedup vs baseline: 1.4414x; 1.0582x over previous
import functools

import jax
import jax.numpy as jnp
from jax import lax
from jax.experimental import pallas as pl
from jax.experimental.pallas import tpu as pltpu
from jax.experimental.pallas import tpu_sc as plsc

D_MODEL = 768
D_FF = 3072
NUM_EXPERTS = 8
N_TOK = 2048
NK = 2 * N_TOK
ROW_TILE = 256
N_TILES = 23
XS_ROWS = N_TILES * ROW_TILE
NW = 32
SLOTS_PER_W = NK // NW
TOK_PER_W = N_TOK // NW


def _router_body(x_ref, gw_ref, pos_ref, w_ref, te_ref):
    xt = x_ref[...]
    logits = lax.dot_general(
        xt.astype(jnp.bfloat16), gw_ref[...].astype(jnp.bfloat16),
        (((1,), (1,)), ((), ())), preferred_element_type=jnp.float32)
    m = jnp.max(logits, axis=-1, keepdims=True)
    ex = jnp.exp(logits - m)
    scores = ex / jnp.sum(ex, axis=-1, keepdims=True)
    iota = lax.broadcasted_iota(jnp.int32, scores.shape, 1)
    m1 = jnp.max(scores, axis=-1, keepdims=True)
    i1 = jnp.min(jnp.where(scores == m1, iota, NUM_EXPERTS), axis=-1,
                 keepdims=True)
    oh1 = iota == i1
    s2 = jnp.where(oh1, -1.0, scores)
    m2 = jnp.max(s2, axis=-1, keepdims=True)
    i2 = jnp.min(jnp.where(s2 == m2, iota, NUM_EXPERTS), axis=-1,
                 keepdims=True)
    oh2 = iota == i2
    denom = m1 + m2 + 1e-9
    oh1f = oh1.astype(jnp.float32)
    oh2f = oh2.astype(jnp.float32)
    ri = lax.broadcasted_iota(jnp.int32, (N_TOK, N_TOK), 0)
    ci = lax.broadcasted_iota(jnp.int32, (N_TOK, N_TOK), 1)
    L = (ri > ci).astype(jnp.float32)
    C0 = lax.dot_general(L, oh1f, (((1,), (0,)), ((), ())),
                         preferred_element_type=jnp.float32)
    C1 = lax.dot_general(L, oh2f, (((1,), (0,)), ((), ())),
                         preferred_element_type=jnp.float32)
    t0 = jnp.sum(oh1f, axis=0, keepdims=True)
    t1 = jnp.sum(oh2f, axis=0, keepdims=True)
    cnt = t0 + t1
    padded = jnp.floor((cnt + (ROW_TILE - 1)) * (1.0 / ROW_TILE)) * ROW_TILE
    ui = lax.broadcasted_iota(jnp.int32, (NUM_EXPERTS, NUM_EXPERTS), 0)
    uj = lax.broadcasted_iota(jnp.int32, (NUM_EXPERTS, NUM_EXPERTS), 1)
    U = (ui < uj).astype(jnp.float32)
    poff = lax.dot_general(padded, U, (((1,), (0,)), ((), ())),
                           preferred_element_type=jnp.float32)
    pos0 = jnp.sum(oh1f * (poff + C0), axis=-1, keepdims=True)
    pos1 = jnp.sum(oh2f * (poff + t0 + C1), axis=-1, keepdims=True)
    pos_ref[...] = jnp.concatenate([pos0, pos1], axis=1).astype(jnp.int32)
    w_ref[...] = jnp.concatenate([m1 / denom, m2 / denom], axis=1)
    eye = (ui == uj).astype(jnp.float32)
    poff_sub = jnp.sum(eye * poff, axis=1, keepdims=True)
    tt = lax.broadcasted_iota(jnp.int32, (NUM_EXPERTS, NW), 1)
    ind = (poff_sub <= tt.astype(jnp.float32) * ROW_TILE).astype(jnp.float32)
    te = jnp.sum(ind, axis=0, keepdims=True) - 1.0
    te_ref[...] = jnp.broadcast_to(te, (NUM_EXPERTS, NW)).astype(jnp.int32)


def _run_router(xf, gate_w):
    return pl.pallas_call(
        _router_body,
        out_shape=(
            jax.ShapeDtypeStruct((N_TOK, 2), jnp.int32),
            jax.ShapeDtypeStruct((N_TOK, 2), jnp.float32),
            jax.ShapeDtypeStruct((NUM_EXPERTS, NW), jnp.int32),
        ),
    )(xf, gate_w)


def _scatter_fn(x_hbm, pos_hbm, xs_hbm, idx_v, rows_v, sem):
    c = lax.axis_index("c")
    s = lax.axis_index("s")
    wid = s * 2 + c
    base = wid * SLOTS_PER_W
    rowbase = lax.rem(wid, 16) * SLOTS_PER_W
    pltpu.sync_copy(pos_hbm.at[pl.ds(base, SLOTS_PER_W)], idx_v)
    pltpu.sync_copy(x_hbm.at[pl.ds(rowbase, SLOTS_PER_W)], rows_v)
    pltpu.async_copy(rows_v, xs_hbm.at[idx_v], sem).wait()


def _run_scatter(xf, pos_flat):
    return pl.kernel(
        _scatter_fn,
        out_type=jax.ShapeDtypeStruct((XS_ROWS, D_MODEL), jnp.float32),
        mesh=plsc.VectorSubcoreMesh(core_axis_name="c", subcore_axis_name="s"),
        scratch_types=[
            pltpu.VMEM((SLOTS_PER_W,), jnp.int32),
            pltpu.VMEM((SLOTS_PER_W, D_MODEL), jnp.float32),
            pltpu.SemaphoreType.DMA,
        ],
    )(xf, pos_flat)


def _ffn_body(te_ref, xs_ref, w1_ref, w2_ref, ys_ref):
    h = lax.dot_general(xs_ref[...], w1_ref[0], (((1,), (1,)), ((), ())),
                        preferred_element_type=jnp.float32)
    h = jnp.maximum(h, 0.0)
    ys_ref[...] = lax.dot_general(h, w2_ref[0], (((1,), (1,)), ((), ())),
                                  preferred_element_type=jnp.float32)


def _run_ffn(te, xs, w1, w2):
    grid_spec = pltpu.PrefetchScalarGridSpec(
        num_scalar_prefetch=1,
        grid=(N_TILES,),
        in_specs=[
            pl.BlockSpec((ROW_TILE, D_MODEL), lambda t, te: (t, 0)),
            pl.BlockSpec((1, D_FF, D_MODEL), lambda t, te: (te[0, t], 0, 0)),
            pl.BlockSpec((1, D_MODEL, D_FF), lambda t, te: (te[0, t], 0, 0)),
        ],
        out_specs=pl.BlockSpec((ROW_TILE, D_MODEL), lambda t, te: (t, 0)),
    )
    return pl.pallas_call(
        _ffn_body,
        grid_spec=grid_spec,
        out_shape=jax.ShapeDtypeStruct((XS_ROWS, D_MODEL), jnp.float32),
    )(te, xs, w1, w2)


def _gather_fn(ys_hbm, pos_hbm, out0_hbm, out1_hbm, idx0, idx1, buf0, buf1,
               sem, sem2):
    c = lax.axis_index("c")
    s = lax.axis_index("s")
    wid = s * 2 + c
    nb = wid * TOK_PER_W
    pltpu.sync_copy(pos_hbm.at[pl.ds(nb, TOK_PER_W)], idx0)
    pltpu.sync_copy(pos_hbm.at[pl.ds(N_TOK + nb, TOK_PER_W)], idx1)
    cp0 = pltpu.async_copy(ys_hbm.at[idx0], buf0, sem)
    cp1 = pltpu.async_copy(ys_hbm.at[idx1], buf1, sem2)
    cp0.wait()
    cp1.wait()
    pltpu.sync_copy(buf0, out0_hbm.at[pl.ds(nb, TOK_PER_W)])
    pltpu.sync_copy(buf1, out1_hbm.at[pl.ds(nb, TOK_PER_W)])


def _run_gather(ys, pos_flat):
    return pl.kernel(
        _gather_fn,
        out_type=(
            jax.ShapeDtypeStruct((N_TOK, D_MODEL), jnp.float32),
            jax.ShapeDtypeStruct((N_TOK, D_MODEL), jnp.float32),
        ),
        mesh=plsc.VectorSubcoreMesh(core_axis_name="c", subcore_axis_name="s"),
        scratch_types=[
            pltpu.VMEM((TOK_PER_W,), jnp.int32),
            pltpu.VMEM((TOK_PER_W,), jnp.int32),
            pltpu.VMEM((TOK_PER_W, D_MODEL), jnp.float32),
            pltpu.VMEM((TOK_PER_W, D_MODEL), jnp.float32),
            pltpu.SemaphoreType.DMA,
            pltpu.SemaphoreType.DMA,
        ],
    )(ys, pos_flat)


def _wsum_body(y0_ref, y1_ref, wc_ref, out_ref):
    wc = wc_ref[...]
    out_ref[...] = (wc[:, 0:1] * y0_ref[...] + wc[:, 1:2] * y1_ref[...])


def _run_wsum(y0, y1, wc):
    return pl.pallas_call(
        _wsum_body,
        out_shape=jax.ShapeDtypeStruct((N_TOK, D_MODEL), jnp.float32),
    )(y0, y1, wc)


@jax.jit
def kernel(x, gate_w, w1, w2):
    B, T, H = x.shape
    xf = x.reshape(B * T, H)
    pos2, w2c, te = _run_router(xf, gate_w)
    pos_flat = pos2.T.reshape(NK)
    xs = _run_scatter(xf, pos_flat)
    ys = _run_ffn(te, xs, w1, w2)
    y0, y1 = _run_gather(ys, pos_flat)
    out = _run_wsum(y0, y1, w2c)
    return out.reshape(B, T, H)


# speedup vs baseline: 1.4462x; 1.0033x over previous
import functools

import jax
import jax.numpy as jnp
from jax import lax
from jax.experimental import pallas as pl
from jax.experimental.pallas import tpu as pltpu
from jax.experimental.pallas import tpu_sc as plsc

D_MODEL = 768
D_FF = 3072
NUM_EXPERTS = 8
N_TOK = 2048
NK = 2 * N_TOK
ROW_TILE = 256
N_TILES = 23
XS_ROWS = N_TILES * ROW_TILE
NW = 32
SLOTS_PER_W = NK // NW
TOK_PER_W = N_TOK // NW


def _router_body(x_ref, gw_ref, pos_ref, xw_ref, te_ref):
    xt = x_ref[...]
    logits = lax.dot_general(
        xt.astype(jnp.bfloat16), gw_ref[...].astype(jnp.bfloat16),
        (((1,), (1,)), ((), ())), preferred_element_type=jnp.float32)
    m = jnp.max(logits, axis=-1, keepdims=True)
    ex = jnp.exp(logits - m)
    scores = ex / jnp.sum(ex, axis=-1, keepdims=True)
    iota = lax.broadcasted_iota(jnp.int32, scores.shape, 1)
    m1 = jnp.max(scores, axis=-1, keepdims=True)
    i1 = jnp.min(jnp.where(scores == m1, iota, NUM_EXPERTS), axis=-1,
                 keepdims=True)
    oh1 = iota == i1
    s2 = jnp.where(oh1, -1.0, scores)
    m2 = jnp.max(s2, axis=-1, keepdims=True)
    i2 = jnp.min(jnp.where(s2 == m2, iota, NUM_EXPERTS), axis=-1,
                 keepdims=True)
    oh2 = iota == i2
    denom = m1 + m2 + 1e-9
    oh1f = oh1.astype(jnp.float32)
    oh2f = oh2.astype(jnp.float32)
    ri = lax.broadcasted_iota(jnp.int32, (N_TOK, N_TOK), 0)
    ci = lax.broadcasted_iota(jnp.int32, (N_TOK, N_TOK), 1)
    L = (ri > ci).astype(jnp.float32)
    C0 = lax.dot_general(L, oh1f, (((1,), (0,)), ((), ())),
                         preferred_element_type=jnp.float32)
    C1 = lax.dot_general(L, oh2f, (((1,), (0,)), ((), ())),
                         preferred_element_type=jnp.float32)
    t0 = jnp.sum(oh1f, axis=0, keepdims=True)
    t1 = jnp.sum(oh2f, axis=0, keepdims=True)
    cnt = t0 + t1
    padded = jnp.floor((cnt + (ROW_TILE - 1)) * (1.0 / ROW_TILE)) * ROW_TILE
    ui = lax.broadcasted_iota(jnp.int32, (NUM_EXPERTS, NUM_EXPERTS), 0)
    uj = lax.broadcasted_iota(jnp.int32, (NUM_EXPERTS, NUM_EXPERTS), 1)
    U = (ui < uj).astype(jnp.float32)
    poff = lax.dot_general(padded, U, (((1,), (0,)), ((), ())),
                           preferred_element_type=jnp.float32)
    pos0 = jnp.sum(oh1f * (poff + C0), axis=-1, keepdims=True)
    pos1 = jnp.sum(oh2f * (poff + t0 + C1), axis=-1, keepdims=True)
    pos_ref[...] = jnp.concatenate([pos0, pos1], axis=1).astype(jnp.int32)
    xw_ref[0:N_TOK, :] = xt * (m1 / denom)
    xw_ref[N_TOK:NK, :] = xt * (m2 / denom)
    eye = (ui == uj).astype(jnp.float32)
    poff_sub = jnp.sum(eye * poff, axis=1, keepdims=True)
    tt = lax.broadcasted_iota(jnp.int32, (NUM_EXPERTS, NW), 1)
    ind = (poff_sub <= tt.astype(jnp.float32) * ROW_TILE).astype(jnp.float32)
    te = jnp.sum(ind, axis=0, keepdims=True) - 1.0
    te_ref[...] = jnp.broadcast_to(te, (NUM_EXPERTS, NW)).astype(jnp.int32)


def _run_router(xf, gate_w):
    return pl.pallas_call(
        _router_body,
        out_shape=(
            jax.ShapeDtypeStruct((N_TOK, 2), jnp.int32),
            jax.ShapeDtypeStruct((NK, D_MODEL), jnp.float32),
            jax.ShapeDtypeStruct((NUM_EXPERTS, NW), jnp.int32),
        ),
    )(xf, gate_w)


def _scatter_fn(x_hbm, pos_hbm, xs_hbm, idx_v, rows_v, sem):
    c = lax.axis_index("c")
    s = lax.axis_index("s")
    wid = s * 2 + c
    base = wid * SLOTS_PER_W
    pltpu.sync_copy(pos_hbm.at[pl.ds(base, SLOTS_PER_W)], idx_v)
    pltpu.sync_copy(x_hbm.at[pl.ds(base, SLOTS_PER_W)], rows_v)
    pltpu.async_copy(rows_v, xs_hbm.at[idx_v], sem).wait()


def _run_scatter(xf, pos_flat):
    return pl.kernel(
        _scatter_fn,
        out_type=jax.ShapeDtypeStruct((XS_ROWS, D_MODEL), jnp.float32),
        mesh=plsc.VectorSubcoreMesh(core_axis_name="c", subcore_axis_name="s"),
        scratch_types=[
            pltpu.VMEM((SLOTS_PER_W,), jnp.int32),
            pltpu.VMEM((SLOTS_PER_W, D_MODEL), jnp.float32),
            pltpu.SemaphoreType.DMA,
        ],
    )(xf, pos_flat)


def _ffn_body(te_ref, xs_ref, w1_ref, w2_ref, ys_ref):
    h = lax.dot_general(xs_ref[...], w1_ref[0], (((1,), (1,)), ((), ())),
                        preferred_element_type=jnp.float32)
    h = jnp.maximum(h, 0.0)
    ys_ref[...] = lax.dot_general(h, w2_ref[0], (((1,), (1,)), ((), ())),
                                  preferred_element_type=jnp.float32)


def _run_ffn(te, xs, w1, w2):
    grid_spec = pltpu.PrefetchScalarGridSpec(
        num_scalar_prefetch=1,
        grid=(N_TILES,),
        in_specs=[
            pl.BlockSpec((ROW_TILE, D_MODEL), lambda t, te: (t, 0)),
            pl.BlockSpec((1, D_FF, D_MODEL), lambda t, te: (te[0, t], 0, 0)),
            pl.BlockSpec((1, D_MODEL, D_FF), lambda t, te: (te[0, t], 0, 0)),
        ],
        out_specs=pl.BlockSpec((ROW_TILE, D_MODEL), lambda t, te: (t, 0)),
    )
    return pl.pallas_call(
        _ffn_body,
        grid_spec=grid_spec,
        out_shape=jax.ShapeDtypeStruct((XS_ROWS, D_MODEL), jnp.float32),
    )(te, xs, w1, w2)


def _gather_fn(ys_hbm, pos_hbm, out0_hbm, out1_hbm, idx0, idx1, buf0, buf1,
               sem, sem2):
    c = lax.axis_index("c")
    s = lax.axis_index("s")
    wid = s * 2 + c
    nb = wid * TOK_PER_W
    pltpu.sync_copy(pos_hbm.at[pl.ds(nb, TOK_PER_W)], idx0)
    pltpu.sync_copy(pos_hbm.at[pl.ds(N_TOK + nb, TOK_PER_W)], idx1)
    cp0 = pltpu.async_copy(ys_hbm.at[idx0], buf0, sem)
    cp1 = pltpu.async_copy(ys_hbm.at[idx1], buf1, sem2)
    cp0.wait()
    cp1.wait()
    pltpu.sync_copy(buf0, out0_hbm.at[pl.ds(nb, TOK_PER_W)])
    pltpu.sync_copy(buf1, out1_hbm.at[pl.ds(nb, TOK_PER_W)])


def _run_gather(ys, pos_flat):
    return pl.kernel(
        _gather_fn,
        out_type=(
            jax.ShapeDtypeStruct((N_TOK, D_MODEL), jnp.float32),
            jax.ShapeDtypeStruct((N_TOK, D_MODEL), jnp.float32),
        ),
        mesh=plsc.VectorSubcoreMesh(core_axis_name="c", subcore_axis_name="s"),
        scratch_types=[
            pltpu.VMEM((TOK_PER_W,), jnp.int32),
            pltpu.VMEM((TOK_PER_W,), jnp.int32),
            pltpu.VMEM((TOK_PER_W, D_MODEL), jnp.float32),
            pltpu.VMEM((TOK_PER_W, D_MODEL), jnp.float32),
            pltpu.SemaphoreType.DMA,
            pltpu.SemaphoreType.DMA,
        ],
    )(ys, pos_flat)


def _add_body(y0_ref, y1_ref, out_ref):
    out_ref[...] = y0_ref[...] + y1_ref[...]


def _run_add(y0, y1):
    return pl.pallas_call(
        _add_body,
        out_shape=jax.ShapeDtypeStruct((N_TOK, D_MODEL), jnp.float32),
    )(y0, y1)


@jax.jit
def kernel(x, gate_w, w1, w2):
    B, T, H = x.shape
    xf = x.reshape(B * T, H)
    pos2, xw, te = _run_router(xf, gate_w)
    pos_flat = pos2.T.reshape(NK)
    xs = _run_scatter(xw, pos_flat)
    ys = _run_ffn(te, xs, w1, w2)
    y0, y1 = _run_gather(ys, pos_flat)
    out = _run_add(y0, y1)
    return out.reshape(B, T, H)
